# baseline (speedup 1.0000x reference)
import functools

import jax
import jax.numpy as jnp
from jax import lax
from jax.experimental import pallas as pl
from jax.experimental.pallas import tpu as pltpu
from jax.experimental.pallas import tpu_sc as plsc

L = 16
NC, NS = 2, 16
NW = NC * NS


def _round_up(x, m):
    return (x + m - 1) // m * m


def _masked_softmax_tc(ptr, relmask, travvec, W):
    B = ptr.shape[0]
    R = W.shape[0]

    def body(ptr_ref, relmask_ref, travvec_ref, w_ref, p_ref):
        rb = lax.dot_general(
            ptr_ref[...], relmask_ref[...], (((1,), (1,)), ((), ())),
            preferred_element_type=jnp.float32) > 0
        y = lax.dot_general(
            travvec_ref[...], w_ref[...], (((1,), (1,)), ((), ())),
            preferred_element_type=jnp.float32)
        ymax = jnp.max(jnp.where(rb, y, -jnp.inf), axis=1, keepdims=True)
        exps = jnp.where(rb, jnp.exp(y - ymax), 0.0)
        p_ref[...] = exps / jnp.sum(exps, axis=1, keepdims=True)

    return pl.pallas_call(
        body, out_shape=jax.ShapeDtypeStruct((B, R), jnp.float32),
    )(ptr, relmask, travvec, W)


def _segmax_sc(ptr, p, sr_pad, sstart, sfan, sperm, sgmax, cbase,
               *, nseg, cseg, fmax, wcap):
    B, E = ptr.shape
    R = p.shape[1]
    nchunk = nseg // cseg
    grp = cseg // L
    mesh = plsc.VectorSubcoreMesh(
        core_axis_name="c", subcore_axis_name="s",
        num_cores=NC, num_subcores=NS)

    @functools.partial(
        pl.kernel,
        out_type=jax.ShapeDtypeStruct((B, nseg), jnp.float32),
        mesh=mesh,
        compiler_params=pltpu.CompilerParams(needs_layout_passes=False),
        scratch_types=[
            pltpu.VMEM((E,), jnp.float32),
            pltpu.VMEM((R,), jnp.float32),
            pltpu.VMEM((nseg,), jnp.int32),
            pltpu.VMEM((nseg,), jnp.int32),
            pltpu.VMEM((nseg,), jnp.int32),
            pltpu.VMEM((sgmax.shape[0],), jnp.int32),
            pltpu.VMEM((cbase.shape[0],), jnp.int32),
            pltpu.VMEM((wcap,), jnp.int32),
            pltpu.VMEM((wcap,), jnp.int32),
            pltpu.VMEM((nseg,), jnp.float32),
            pltpu.SemaphoreType.DMA,
            pltpu.SemaphoreType.DMA,
        ],
    )
    def k(ptr_hbm, p_hbm, sr_hbm, sstart_hbm, sfan_hbm, sperm_hbm,
          sgmax_hbm, cbase_hbm, out_hbm,
          ptr_v, p_v, sstart_v, sfan_v, sperm_v, sgmax_v, cbase_v,
          win0_v, win1_v, out_v, sem0, sem1):
        b = lax.axis_index("s") * NC + lax.axis_index("c")
        pltpu.sync_copy(ptr_hbm.at[b], ptr_v)
        pltpu.sync_copy(p_hbm.at[b], p_v)
        pltpu.sync_copy(sstart_hbm, sstart_v)
        pltpu.sync_copy(sfan_hbm, sfan_v)
        pltpu.sync_copy(sperm_hbm, sperm_v)
        pltpu.sync_copy(sgmax_hbm, sgmax_v)
        pltpu.sync_copy(cbase_hbm, cbase_v)

        def start_dma(c, win_v, sem):
            base = cbase_v[pl.ds(c, L)][0]
            pltpu.async_copy(
                sr_hbm.at[pl.ds(pl.multiple_of(base, 8), wcap)], win_v,
                sem)

        def wait_dma(win_v, sem):
            pltpu.make_async_copy(
                sr_hbm.at[pl.ds(0, wcap)], win_v, sem).wait()

        def compute(c, win_v):
            base = cbase_v[pl.ds(c, L)][0]

            @plsc.parallel_loop(0, grp, unroll=2)
            def seg_body(g):
                jb = c * cseg + g * L
                gg = c * grp + g
                loc = sstart_v[pl.ds(jb, L)] - base
                fanv = sfan_v[pl.ds(jb, L)]
                perm = sperm_v[pl.ds(jb, L)]
                kmax = sgmax_v[pl.ds(gg, L)][0]

                @plsc.parallel_loop(0, kmax, unroll=8,
                                    carry=jnp.zeros((L,), jnp.float32))
                def max_body(kk, acc):
                    srv = plsc.load_gather(win_v, [loc + kk])
                    s = jnp.bitwise_and(srv, jnp.int32(16383))
                    r = jnp.right_shift(srv, jnp.int32(14))
                    prod = plsc.load_gather(ptr_v, [s]) * \
                        plsc.load_gather(p_v, [r])
                    return jnp.maximum(
                        acc, jnp.where(fanv > kk, prod, 0.0))
                plsc.store_scatter(out_v, [perm], max_body)

        start_dma(0, win0_v, sem0)

        def pair_body(c2, _):
            c0 = c2 * 2
            start_dma(c0 + 1, win1_v, sem1)
            wait_dma(win0_v, sem0)
            compute(c0, win0_v)
            start_dma(c0 + 2, win0_v, sem0)
            wait_dma(win1_v, sem1)
            compute(c0 + 1, win1_v)
            return 0
        lax.fori_loop(0, nchunk // 2, pair_body, 0)
        wait_dma(win0_v, sem0)
        compute(nchunk - 1, win0_v)
        pltpu.sync_copy(out_v, out_hbm.at[b])

    return k(ptr, p, sr_pad, sstart, sfan, sperm, sgmax, cbase)


def kernel(ptr, travvec, W, tensor_idx, relmask, groupbys, groupbymasks):
    B, E = ptr.shape
    T = tensor_idx.shape[0]
    assert B == NW

    p = _masked_softmax_tc(ptr, relmask, travvec, W)

    start = jnp.concatenate(
        [g[:, 0].astype(jnp.int32) for g in groupbys])
    fan = jnp.concatenate(
        [m.sum(axis=1).astype(jnp.int32) for m in groupbymasks])
    nseg = sum(g.shape[0] for g in groupbys)
    fmax = max(g.shape[1] for g in groupbys)

    cseg = 400
    assert nseg % cseg == 0 and cseg % L == 0
    wcap = _round_up(cseg * fmax + fmax + 8, L)

    assert E <= (1 << 14)
    sr = jnp.bitwise_or(
        jnp.left_shift(tensor_idx[:, 0].astype(jnp.int32), 14),
        tensor_idx[:, 1].astype(jnp.int32))
    sr_pad = jnp.pad(sr, (0, _round_up(T + wcap, L) - T))

    nchunk = nseg // cseg
    grp = cseg // L
    cbase = (start[::cseg] // 8 * 8).astype(jnp.int32)
    order = jnp.argsort(fan.reshape(nchunk, cseg), axis=1)
    sstart = jnp.take_along_axis(
        start.reshape(nchunk, cseg), order, axis=1).reshape(-1)
    sfan = jnp.take_along_axis(
        fan.reshape(nchunk, cseg), order, axis=1).reshape(-1)
    sperm = (order + jnp.arange(nchunk, dtype=jnp.int32)[:, None] * cseg
             ).astype(jnp.int32).reshape(-1)
    ngrp = nseg // L
    sgmax = jnp.max(sfan.reshape(ngrp, L), axis=1).astype(jnp.int32)
    sgmax = jnp.pad(sgmax, (0, _round_up(ngrp + L, L) - ngrp))
    cbase = jnp.pad(cbase, (0, _round_up(nchunk + L, L) - nchunk))

    return _segmax_sc(ptr, p, sr_pad, sstart, sfan, sperm, sgmax, cbase,
                      nseg=nseg, cseg=cseg, fmax=fmax, wcap=wcap)


# speedup vs baseline: 1.0641x; 1.0641x over previous
import functools

import jax
import jax.numpy as jnp
from jax import lax
from jax.experimental import pallas as pl
from jax.experimental.pallas import tpu as pltpu
from jax.experimental.pallas import tpu_sc as plsc

L = 16
NC, NS = 2, 16
NW = NC * NS


def _round_up(x, m):
    return (x + m - 1) // m * m


def _masked_softmax_tc(ptr, relmask, travvec, W):
    B = ptr.shape[0]
    R = W.shape[0]

    def body(ptr_ref, relmask_ref, travvec_ref, w_ref, p_ref):
        rb = lax.dot_general(
            ptr_ref[...], relmask_ref[...], (((1,), (1,)), ((), ())),
            preferred_element_type=jnp.float32) > 0
        y = lax.dot_general(
            travvec_ref[...], w_ref[...], (((1,), (1,)), ((), ())),
            preferred_element_type=jnp.float32)
        ymax = jnp.max(jnp.where(rb, y, -jnp.inf), axis=1, keepdims=True)
        exps = jnp.where(rb, jnp.exp(y - ymax), 0.0)
        p_ref[...] = exps / jnp.sum(exps, axis=1, keepdims=True)

    return pl.pallas_call(
        body, out_shape=jax.ShapeDtypeStruct((B, R), jnp.float32),
    )(ptr, relmask, travvec, W)


def _segmax_sc(ptr, p, sr_pad, sstart, sfan, sperm, sgmax, cbase,
               *, nseg, cseg, fmax, wcap):
    B, E = ptr.shape
    R = p.shape[1]
    nchunk = nseg // cseg
    grp = cseg // L
    mesh = plsc.VectorSubcoreMesh(
        core_axis_name="c", subcore_axis_name="s",
        num_cores=NC, num_subcores=NS)

    @functools.partial(
        pl.kernel,
        out_type=jax.ShapeDtypeStruct((B, nseg), jnp.float32),
        mesh=mesh,
        compiler_params=pltpu.CompilerParams(needs_layout_passes=False),
        scratch_types=[
            pltpu.VMEM((E,), jnp.float32),
            pltpu.VMEM((R,), jnp.float32),
            pltpu.VMEM((nseg,), jnp.int32),
            pltpu.VMEM((nseg,), jnp.int32),
            pltpu.VMEM((nseg,), jnp.int32),
            pltpu.VMEM((sgmax.shape[0],), jnp.int32),
            pltpu.VMEM((cbase.shape[0],), jnp.int32),
            pltpu.VMEM((wcap,), jnp.int32),
            pltpu.VMEM((wcap,), jnp.int32),
            pltpu.VMEM((nseg,), jnp.float32),
            pltpu.SemaphoreType.DMA,
            pltpu.SemaphoreType.DMA,
        ],
    )
    def k(ptr_hbm, p_hbm, sr_hbm, sstart_hbm, sfan_hbm, sperm_hbm,
          sgmax_hbm, cbase_hbm, out_hbm,
          ptr_v, p_v, sstart_v, sfan_v, sperm_v, sgmax_v, cbase_v,
          win0_v, win1_v, out_v, sem0, sem1):
        b = lax.axis_index("s") * NC + lax.axis_index("c")
        lanes = lax.broadcasted_iota(jnp.int32, (L,), 0)
        pltpu.sync_copy(ptr_hbm.at[b], ptr_v)
        pltpu.sync_copy(p_hbm.at[b], p_v)
        pltpu.sync_copy(sstart_hbm, sstart_v)
        pltpu.sync_copy(sfan_hbm, sfan_v)
        pltpu.sync_copy(sperm_hbm, sperm_v)
        pltpu.sync_copy(sgmax_hbm, sgmax_v)
        pltpu.sync_copy(cbase_hbm, cbase_v)

        def start_dma(c, win_v, sem):
            base = cbase_v[pl.ds(c, L)][0]
            pltpu.async_copy(
                sr_hbm.at[pl.ds(pl.multiple_of(base, 8), wcap)], win_v,
                sem)

        def wait_dma(win_v, sem):
            pltpu.make_async_copy(
                sr_hbm.at[pl.ds(0, wcap)], win_v, sem).wait()

        def compute(c, win_v):
            base = cbase_v[pl.ds(c, L)][0]

            @plsc.parallel_loop(0, grp, unroll=2)
            def seg_body(g):
                jb = c * cseg + g * L
                gg = c * grp + g
                loc = sstart_v[pl.ds(jb, L)] - base
                fanv = sfan_v[pl.ds(jb, L)]
                perm = sperm_v[pl.ds(jb, L)]
                kmax = sgmax_v[pl.ds(gg, L)][0]

                @plsc.parallel_loop(0, kmax, unroll=4,
                                    carry=jnp.zeros((L,), jnp.float32))
                def max_body(kk, acc):
                    srv = plsc.load_gather(win_v, [loc + kk])
                    s = jnp.bitwise_and(srv, jnp.int32(16383))
                    r16 = jnp.bitwise_and(jnp.right_shift(srv, jnp.int32(10)),
                                          jnp.int32(-16))
                    prod = plsc.load_gather(ptr_v, [s]) * \
                        plsc.load_gather(p_v, [jnp.bitwise_or(r16, lanes)])
                    return jnp.maximum(
                        acc, jnp.where(fanv > kk, prod, 0.0))
                plsc.store_scatter(out_v, [perm], max_body)

        start_dma(0, win0_v, sem0)

        def pair_body(c2, _):
            c0 = c2 * 2
            start_dma(c0 + 1, win1_v, sem1)
            wait_dma(win0_v, sem0)
            compute(c0, win0_v)
            start_dma(c0 + 2, win0_v, sem0)
            wait_dma(win1_v, sem1)
            compute(c0 + 1, win1_v)
            return 0
        lax.fori_loop(0, nchunk // 2, pair_body, 0)
        wait_dma(win0_v, sem0)
        compute(nchunk - 1, win0_v)
        pltpu.sync_copy(out_v, out_hbm.at[b])

    return k(ptr, p, sr_pad, sstart, sfan, sperm, sgmax, cbase)


def kernel(ptr, travvec, W, tensor_idx, relmask, groupbys, groupbymasks):
    B, E = ptr.shape
    T = tensor_idx.shape[0]
    assert B == NW

    p = _masked_softmax_tc(ptr, relmask, travvec, W)
    p = jnp.repeat(p, L, axis=1)

    start = jnp.concatenate(
        [g[:, 0].astype(jnp.int32) for g in groupbys])
    fan = jnp.concatenate(
        [m.sum(axis=1).astype(jnp.int32) for m in groupbymasks])
    nseg = sum(g.shape[0] for g in groupbys)
    fmax = max(g.shape[1] for g in groupbys)

    cseg = 400
    assert nseg % cseg == 0 and cseg % L == 0
    wcap = _round_up(cseg * fmax + fmax + 8, L)

    assert E <= (1 << 14)
    sr = jnp.bitwise_or(
        jnp.left_shift(tensor_idx[:, 0].astype(jnp.int32), 14),
        tensor_idx[:, 1].astype(jnp.int32))
    sr_pad = jnp.pad(sr, (0, _round_up(T + wcap, L) - T))

    nchunk = nseg // cseg
    grp = cseg // L
    cbase = (start[::cseg] // 8 * 8).astype(jnp.int32)
    order = jnp.argsort(fan.reshape(nchunk, cseg), axis=1)
    sstart = jnp.take_along_axis(
        start.reshape(nchunk, cseg), order, axis=1).reshape(-1)
    sfan = jnp.take_along_axis(
        fan.reshape(nchunk, cseg), order, axis=1).reshape(-1)
    sperm = (order + jnp.arange(nchunk, dtype=jnp.int32)[:, None] * cseg
             ).astype(jnp.int32).reshape(-1)
    ngrp = nseg // L
    sgmax = jnp.max(sfan.reshape(ngrp, L), axis=1).astype(jnp.int32)
    sgmax = jnp.pad(sgmax, (0, _round_up(ngrp + L, L) - ngrp))
    cbase = jnp.pad(cbase, (0, _round_up(nchunk + L, L) - nchunk))

    return _segmax_sc(ptr, p, sr_pad, sstart, sfan, sperm, sgmax, cbase,
                      nseg=nseg, cseg=cseg, fmax=fmax, wcap=wcap)


# speedup vs baseline: 1.2322x; 1.1579x over previous
import functools

import jax
import jax.numpy as jnp
from jax import lax
from jax.experimental import pallas as pl
from jax.experimental.pallas import tpu as pltpu
from jax.experimental.pallas import tpu_sc as plsc

L = 16
NC, NS = 2, 16
NW = NC * NS


def _round_up(x, m):
    return (x + m - 1) // m * m


def _masked_softmax_tc(ptr, relmask, travvec, W):
    B = ptr.shape[0]
    R = W.shape[0]

    def body(ptr_ref, relmask_ref, travvec_ref, w_ref, p_ref):
        rb = lax.dot_general(
            ptr_ref[...], relmask_ref[...], (((1,), (1,)), ((), ())),
            preferred_element_type=jnp.float32) > 0
        y = lax.dot_general(
            travvec_ref[...], w_ref[...], (((1,), (1,)), ((), ())),
            preferred_element_type=jnp.float32)
        ymax = jnp.max(jnp.where(rb, y, -jnp.inf), axis=1, keepdims=True)
        exps = jnp.where(rb, jnp.exp(y - ymax), 0.0)
        p_ref[...] = exps / jnp.sum(exps, axis=1, keepdims=True)

    return pl.pallas_call(
        body, out_shape=jax.ShapeDtypeStruct((B, R), jnp.float32),
    )(ptr, relmask, travvec, W)


def _segmax_sc(ptr, p, sr_pad, sstart, sfan, sperm, sgmax, cbase,
               *, nseg, cseg, fmax, wcap):
    B, E = ptr.shape
    R = p.shape[1]
    nchunk = nseg // cseg
    grp = cseg // L
    mesh = plsc.VectorSubcoreMesh(
        core_axis_name="c", subcore_axis_name="s",
        num_cores=NC, num_subcores=NS)

    @functools.partial(
        pl.kernel,
        out_type=jax.ShapeDtypeStruct((B, nseg), jnp.float32),
        mesh=mesh,
        compiler_params=pltpu.CompilerParams(needs_layout_passes=False),
        scratch_types=[
            pltpu.VMEM((E,), jnp.float32),
            pltpu.VMEM((R,), jnp.float32),
            pltpu.VMEM((nseg,), jnp.int32),
            pltpu.VMEM((nseg,), jnp.int32),
            pltpu.VMEM((nseg,), jnp.int32),
            pltpu.VMEM((sgmax.shape[0],), jnp.int32),
            pltpu.VMEM((cbase.shape[0],), jnp.int32),
            pltpu.VMEM((wcap,), jnp.int32),
            pltpu.VMEM((wcap,), jnp.int32),
            pltpu.VMEM((nseg,), jnp.float32),
            pltpu.SemaphoreType.DMA,
            pltpu.SemaphoreType.DMA,
        ],
    )
    def k(ptr_hbm, p_hbm, sr_hbm, sstart_hbm, sfan_hbm, sperm_hbm,
          sgmax_hbm, cbase_hbm, out_hbm,
          ptr_v, p_v, sstart_v, sfan_v, sperm_v, sgmax_v, cbase_v,
          win0_v, win1_v, out_v, sem0, sem1):
        b = lax.axis_index("s") * NC + lax.axis_index("c")
        pltpu.sync_copy(ptr_hbm.at[b], ptr_v)
        pltpu.sync_copy(p_hbm.at[b], p_v)
        pltpu.sync_copy(sstart_hbm, sstart_v)
        pltpu.sync_copy(sfan_hbm, sfan_v)
        pltpu.sync_copy(sperm_hbm, sperm_v)
        pltpu.sync_copy(sgmax_hbm, sgmax_v)
        pltpu.sync_copy(cbase_hbm, cbase_v)

        def start_dma(c, win_v, sem):
            base = cbase_v[pl.ds(c, L)][0]
            pltpu.async_copy(
                sr_hbm.at[pl.ds(pl.multiple_of(base, 8), wcap)], win_v,
                sem)

        def wait_dma(win_v, sem):
            pltpu.make_async_copy(
                sr_hbm.at[pl.ds(0, wcap)], win_v, sem).wait()

        def compute(c, win_v):
            base = cbase_v[pl.ds(c, L)][0]

            def seg_block(kk, loc, fanv, acc):
                srv = plsc.load_gather(win_v, [loc + kk])
                s = jnp.bitwise_and(srv, jnp.int32(16383))
                r = jnp.right_shift(srv, jnp.int32(14))
                prod = plsc.load_gather(ptr_v, [s]) * \
                    plsc.load_gather(p_v, [r])
                return jnp.maximum(acc, jnp.where(fanv > kk, prod, 0.0))

            @plsc.parallel_loop(0, grp // 2, unroll=1)
            def seg_body(pg):
                jb = c * cseg + pg * (2 * L)
                gg = c * grp + pg * 2
                loc_a = sstart_v[pl.ds(jb, L)] - base
                fan_a = sfan_v[pl.ds(jb, L)]
                perm_a = sperm_v[pl.ds(jb, L)]
                loc_b = sstart_v[pl.ds(jb + L, L)] - base
                fan_b = sfan_v[pl.ds(jb + L, L)]
                perm_b = sperm_v[pl.ds(jb + L, L)]
                kmax = sgmax_v[pl.ds(gg + 1, L)][0]

                @plsc.parallel_loop(0, kmax, unroll=4,
                                    carry=(jnp.zeros((L,), jnp.float32),
                                           jnp.zeros((L,), jnp.float32)))
                def max_body(kk, acc):
                    acc_a, acc_b = acc
                    return (seg_block(kk, loc_a, fan_a, acc_a),
                            seg_block(kk, loc_b, fan_b, acc_b))
                acc_a, acc_b = max_body
                plsc.store_scatter(out_v, [perm_a], acc_a)
                plsc.store_scatter(out_v, [perm_b], acc_b)

            jb = c * cseg + (grp - 1) * L
            gg = c * grp + (grp - 1)
            loc = sstart_v[pl.ds(jb, L)] - base
            fanv = sfan_v[pl.ds(jb, L)]
            perm = sperm_v[pl.ds(jb, L)]
            kmax = sgmax_v[pl.ds(gg, L)][0]

            @plsc.parallel_loop(0, kmax, unroll=4,
                                carry=jnp.zeros((L,), jnp.float32))
            def tail_body(kk, acc):
                return seg_block(kk, loc, fanv, acc)
            plsc.store_scatter(out_v, [perm], tail_body)

        start_dma(0, win0_v, sem0)

        def pair_body(c2, _):
            c0 = c2 * 2
            start_dma(c0 + 1, win1_v, sem1)
            wait_dma(win0_v, sem0)
            compute(c0, win0_v)
            start_dma(c0 + 2, win0_v, sem0)
            wait_dma(win1_v, sem1)
            compute(c0 + 1, win1_v)
            return 0
        lax.fori_loop(0, nchunk // 2, pair_body, 0)
        wait_dma(win0_v, sem0)
        compute(nchunk - 1, win0_v)
        pltpu.sync_copy(out_v, out_hbm.at[b])

    return k(ptr, p, sr_pad, sstart, sfan, sperm, sgmax, cbase)


def kernel(ptr, travvec, W, tensor_idx, relmask, groupbys, groupbymasks):
    B, E = ptr.shape
    T = tensor_idx.shape[0]
    assert B == NW

    p = _masked_softmax_tc(ptr, relmask, travvec, W)

    start = jnp.concatenate(
        [g[:, 0].astype(jnp.int32) for g in groupbys])
    fan = jnp.concatenate(
        [m.sum(axis=1).astype(jnp.int32) for m in groupbymasks])
    nseg = sum(g.shape[0] for g in groupbys)
    fmax = max(g.shape[1] for g in groupbys)

    cseg = 400
    assert nseg % cseg == 0 and cseg % L == 0
    wcap = _round_up(cseg * fmax + fmax + 8, L)

    assert E <= (1 << 14)
    sr = jnp.bitwise_or(
        jnp.left_shift(tensor_idx[:, 0].astype(jnp.int32), 14),
        tensor_idx[:, 1].astype(jnp.int32))
    sr_pad = jnp.pad(sr, (0, _round_up(T + wcap, L) - T))

    nchunk = nseg // cseg
    grp = cseg // L
    cbase = (start[::cseg] // 8 * 8).astype(jnp.int32)
    order = jnp.argsort(fan.reshape(nchunk, cseg), axis=1)
    sstart = jnp.take_along_axis(
        start.reshape(nchunk, cseg), order, axis=1).reshape(-1)
    sfan = jnp.take_along_axis(
        fan.reshape(nchunk, cseg), order, axis=1).reshape(-1)
    sperm = (order + jnp.arange(nchunk, dtype=jnp.int32)[:, None] * cseg
             ).astype(jnp.int32).reshape(-1)
    ngrp = nseg // L
    sgmax = jnp.max(sfan.reshape(ngrp, L), axis=1).astype(jnp.int32)
    sgmax = jnp.pad(sgmax, (0, _round_up(ngrp + L, L) - ngrp))
    cbase = jnp.pad(cbase, (0, _round_up(nchunk + L, L) - nchunk))

    return _segmax_sc(ptr, p, sr_pad, sstart, sfan, sperm, sgmax, cbase,
                      nseg=nseg, cseg=cseg, fmax=fmax, wcap=wcap)


# speedup vs baseline: 1.2953x; 1.0512x over previous
import functools

import jax
import jax.numpy as jnp
from jax import lax
from jax.experimental import pallas as pl
from jax.experimental.pallas import tpu as pltpu
from jax.experimental.pallas import tpu_sc as plsc

L = 16
NC, NS = 2, 16
NW = NC * NS


def _round_up(x, m):
    return (x + m - 1) // m * m


def _masked_softmax_tc(ptr, relmask, travvec, W):
    B = ptr.shape[0]
    R = W.shape[0]

    def body(ptr_ref, relmask_ref, travvec_ref, w_ref, p_ref):
        rb = lax.dot_general(
            ptr_ref[...], relmask_ref[...], (((1,), (1,)), ((), ())),
            preferred_element_type=jnp.float32) > 0
        y = lax.dot_general(
            travvec_ref[...], w_ref[...], (((1,), (1,)), ((), ())),
            preferred_element_type=jnp.float32)
        ymax = jnp.max(jnp.where(rb, y, -jnp.inf), axis=1, keepdims=True)
        exps = jnp.where(rb, jnp.exp(y - ymax), 0.0)
        p_ref[...] = exps / jnp.sum(exps, axis=1, keepdims=True)

    return pl.pallas_call(
        body, out_shape=jax.ShapeDtypeStruct((B, R), jnp.float32),
    )(ptr, relmask, travvec, W)


def _segmax_sc(ptr, p, sr_pad, sstart, sfan, sperm, sgmax, cbase,
               *, nseg, cseg, fmax, wcap):
    B, E = ptr.shape
    R = p.shape[1]
    nchunk = nseg // cseg
    grp = cseg // L
    mesh = plsc.VectorSubcoreMesh(
        core_axis_name="c", subcore_axis_name="s",
        num_cores=NC, num_subcores=NS)

    @functools.partial(
        pl.kernel,
        out_type=jax.ShapeDtypeStruct((B, nseg), jnp.float32),
        mesh=mesh,
        compiler_params=pltpu.CompilerParams(needs_layout_passes=False),
        scratch_types=[
            pltpu.VMEM((E,), jnp.float32),
            pltpu.VMEM((R,), jnp.float32),
            pltpu.VMEM((nseg,), jnp.int32),
            pltpu.VMEM((nseg,), jnp.int32),
            pltpu.VMEM((nseg,), jnp.int32),
            pltpu.VMEM((sgmax.shape[0],), jnp.int32),
            pltpu.VMEM((cbase.shape[0],), jnp.int32),
            pltpu.VMEM((wcap,), jnp.int32),
            pltpu.VMEM((wcap,), jnp.int32),
            pltpu.VMEM((nseg,), jnp.float32),
            pltpu.SemaphoreType.DMA,
            pltpu.SemaphoreType.DMA,
        ],
    )
    def k(ptr_hbm, p_hbm, sr_hbm, sstart_hbm, sfan_hbm, sperm_hbm,
          sgmax_hbm, cbase_hbm, out_hbm,
          ptr_v, p_v, sstart_v, sfan_v, sperm_v, sgmax_v, cbase_v,
          win0_v, win1_v, out_v, sem0, sem1):
        b = lax.axis_index("s") * NC + lax.axis_index("c")
        pltpu.sync_copy(ptr_hbm.at[b], ptr_v)
        pltpu.sync_copy(p_hbm.at[b], p_v)
        pltpu.sync_copy(sstart_hbm, sstart_v)
        pltpu.sync_copy(sfan_hbm, sfan_v)
        pltpu.sync_copy(sperm_hbm, sperm_v)
        pltpu.sync_copy(sgmax_hbm, sgmax_v)
        pltpu.sync_copy(cbase_hbm, cbase_v)

        def start_dma(c, win_v, sem):
            base = cbase_v[pl.ds(c, L)][0]
            pltpu.async_copy(
                sr_hbm.at[pl.ds(pl.multiple_of(base, 8), wcap)], win_v,
                sem)

        def wait_dma(win_v, sem):
            pltpu.make_async_copy(
                sr_hbm.at[pl.ds(0, wcap)], win_v, sem).wait()

        def compute(c, win_v):
            base = cbase_v[pl.ds(c, L)][0]

            def seg_block(kk, loc, fanv, acc):
                srv = plsc.load_gather(win_v, [loc + kk])
                s = jnp.bitwise_and(srv, jnp.int32(16383))
                r = jnp.right_shift(srv, jnp.int32(14))
                prod = plsc.load_gather(ptr_v, [s]) * \
                    plsc.load_gather(p_v, [r])
                return jnp.maximum(acc, jnp.where(fanv > kk, prod, 0.0))

            def macro_group(jb, gg, nsub, unroll):
                locs = [sstart_v[pl.ds(jb + i * L, L)] - base
                        for i in range(nsub)]
                fans = [sfan_v[pl.ds(jb + i * L, L)] for i in range(nsub)]
                perms = [sperm_v[pl.ds(jb + i * L, L)]
                         for i in range(nsub)]
                kmax = sgmax_v[pl.ds(gg + nsub - 1, L)][0]

                zero = jnp.zeros((L,), jnp.float32)
                init = zero if nsub == 1 else tuple(zero
                                                    for _ in range(nsub))

                @plsc.parallel_loop(0, kmax, unroll=unroll, carry=init)
                def max_body(kk, acc):
                    if nsub == 1:
                        return seg_block(kk, locs[0], fans[0], acc)
                    return tuple(
                        seg_block(kk, locs[i], fans[i], acc[i])
                        for i in range(nsub))
                accs = (max_body,) if nsub == 1 else max_body
                for i in range(nsub):
                    plsc.store_scatter(out_v, [perms[i]], accs[i])

            @plsc.parallel_loop(0, grp // 4, unroll=1)
            def seg_body(pg):
                macro_group(c * cseg + pg * (4 * L),
                            c * grp + pg * 4, 4, 2)

            macro_group(c * cseg + (grp - 1) * L,
                        c * grp + (grp - 1), 1, 4)

        start_dma(0, win0_v, sem0)

        def pair_body(c2, _):
            c0 = c2 * 2
            start_dma(c0 + 1, win1_v, sem1)
            wait_dma(win0_v, sem0)
            compute(c0, win0_v)
            start_dma(c0 + 2, win0_v, sem0)
            wait_dma(win1_v, sem1)
            compute(c0 + 1, win1_v)
            return 0
        lax.fori_loop(0, nchunk // 2, pair_body, 0)
        wait_dma(win0_v, sem0)
        compute(nchunk - 1, win0_v)
        pltpu.sync_copy(out_v, out_hbm.at[b])

    return k(ptr, p, sr_pad, sstart, sfan, sperm, sgmax, cbase)


def kernel(ptr, travvec, W, tensor_idx, relmask, groupbys, groupbymasks):
    B, E = ptr.shape
    T = tensor_idx.shape[0]
    assert B == NW

    p = _masked_softmax_tc(ptr, relmask, travvec, W)

    start = jnp.concatenate(
        [g[:, 0].astype(jnp.int32) for g in groupbys])
    fan = jnp.concatenate(
        [m.sum(axis=1).astype(jnp.int32) for m in groupbymasks])
    nseg = sum(g.shape[0] for g in groupbys)
    fmax = max(g.shape[1] for g in groupbys)

    cseg = 400
    assert nseg % cseg == 0 and cseg % L == 0
    wcap = _round_up(cseg * fmax + fmax + 8, L)

    assert E <= (1 << 14)
    sr = jnp.bitwise_or(
        jnp.left_shift(tensor_idx[:, 0].astype(jnp.int32), 14),
        tensor_idx[:, 1].astype(jnp.int32))
    sr_pad = jnp.pad(sr, (0, _round_up(T + wcap, L) - T))

    nchunk = nseg // cseg
    grp = cseg // L
    cbase = (start[::cseg] // 8 * 8).astype(jnp.int32)
    order = jnp.argsort(fan.reshape(nchunk, cseg), axis=1)
    sstart = jnp.take_along_axis(
        start.reshape(nchunk, cseg), order, axis=1).reshape(-1)
    sfan = jnp.take_along_axis(
        fan.reshape(nchunk, cseg), order, axis=1).reshape(-1)
    sperm = (order + jnp.arange(nchunk, dtype=jnp.int32)[:, None] * cseg
             ).astype(jnp.int32).reshape(-1)
    ngrp = nseg // L
    sgmax = jnp.max(sfan.reshape(ngrp, L), axis=1).astype(jnp.int32)
    sgmax = jnp.pad(sgmax, (0, _round_up(ngrp + L, L) - ngrp))
    cbase = jnp.pad(cbase, (0, _round_up(nchunk + L, L) - nchunk))

    return _segmax_sc(ptr, p, sr_pad, sstart, sfan, sperm, sgmax, cbase,
                      nseg=nseg, cseg=cseg, fmax=fmax, wcap=wcap)


# speedup vs baseline: 1.3077x; 1.0096x over previous
import functools

import jax
import jax.numpy as jnp
from jax import lax
from jax.experimental import pallas as pl
from jax.experimental.pallas import tpu as pltpu
from jax.experimental.pallas import tpu_sc as plsc

L = 16
NC, NS = 2, 16
NW = NC * NS


def _round_up(x, m):
    return (x + m - 1) // m * m


def _masked_softmax_tc(ptr, relmask, travvec, W):
    B = ptr.shape[0]
    R = W.shape[0]

    def body(ptr_ref, relmask_ref, travvec_ref, w_ref, p_ref):
        rb = lax.dot_general(
            ptr_ref[...], relmask_ref[...], (((1,), (1,)), ((), ())),
            preferred_element_type=jnp.float32) > 0
        y = lax.dot_general(
            travvec_ref[...], w_ref[...], (((1,), (1,)), ((), ())),
            preferred_element_type=jnp.float32)
        ymax = jnp.max(jnp.where(rb, y, -jnp.inf), axis=1, keepdims=True)
        exps = jnp.where(rb, jnp.exp(y - ymax), 0.0)
        p_ref[...] = exps / jnp.sum(exps, axis=1, keepdims=True)

    return pl.pallas_call(
        body, out_shape=jax.ShapeDtypeStruct((B, R), jnp.float32),
    )(ptr, relmask, travvec, W)


def _segmax_sc(ptr, p, sr_pad, sstart, sfan, sperm, sgmax, cbase,
               *, nseg, cseg, fmax, wcap):
    B, E = ptr.shape
    R = p.shape[1]
    nchunk = nseg // cseg
    grp = cseg // L
    mesh = plsc.VectorSubcoreMesh(
        core_axis_name="c", subcore_axis_name="s",
        num_cores=NC, num_subcores=NS)

    @functools.partial(
        pl.kernel,
        out_type=jax.ShapeDtypeStruct((B, nseg), jnp.float32),
        mesh=mesh,
        compiler_params=pltpu.CompilerParams(needs_layout_passes=False),
        scratch_types=[
            pltpu.VMEM((E,), jnp.float32),
            pltpu.VMEM((R,), jnp.float32),
            pltpu.VMEM((nseg,), jnp.int32),
            pltpu.VMEM((nseg,), jnp.int32),
            pltpu.VMEM((nseg,), jnp.int32),
            pltpu.VMEM((sgmax.shape[0],), jnp.int32),
            pltpu.VMEM((cbase.shape[0],), jnp.int32),
            pltpu.VMEM((wcap,), jnp.int32),
            pltpu.VMEM((wcap,), jnp.int32),
            pltpu.VMEM((nseg,), jnp.float32),
            pltpu.SemaphoreType.DMA,
            pltpu.SemaphoreType.DMA,
        ],
    )
    def k(ptr_hbm, p_hbm, sr_hbm, sstart_hbm, sfan_hbm, sperm_hbm,
          sgmax_hbm, cbase_hbm, out_hbm,
          ptr_v, p_v, sstart_v, sfan_v, sperm_v, sgmax_v, cbase_v,
          win0_v, win1_v, out_v, sem0, sem1):
        b = lax.axis_index("s") * NC + lax.axis_index("c")
        pltpu.sync_copy(ptr_hbm.at[b], ptr_v)
        pltpu.sync_copy(p_hbm.at[b], p_v)
        pltpu.sync_copy(sstart_hbm, sstart_v)
        pltpu.sync_copy(sfan_hbm, sfan_v)
        pltpu.sync_copy(sperm_hbm, sperm_v)
        pltpu.sync_copy(sgmax_hbm, sgmax_v)
        pltpu.sync_copy(cbase_hbm, cbase_v)

        def start_dma(c, win_v, sem):
            base = cbase_v[pl.ds(c, L)][0]
            pltpu.async_copy(
                sr_hbm.at[pl.ds(pl.multiple_of(base, 8), wcap)], win_v,
                sem)

        def wait_dma(win_v, sem):
            pltpu.make_async_copy(
                sr_hbm.at[pl.ds(0, wcap)], win_v, sem).wait()

        def compute(c, win_v):
            base = cbase_v[pl.ds(c, L)][0]

            def seg_block(kk, loc, fanv, acc):
                srv = plsc.load_gather(win_v, [loc + kk])
                s = jnp.bitwise_and(srv, jnp.int32(16383))
                r = jnp.right_shift(srv, jnp.int32(14))
                prod = plsc.load_gather(ptr_v, [s]) * \
                    plsc.load_gather(p_v, [r])
                return jnp.maximum(acc, jnp.where(fanv > kk, prod, 0.0))

            def macro_group(jb, gg, nsub, unroll):
                locs = [sstart_v[pl.ds(jb + i * L, L)] - base
                        for i in range(nsub)]
                fans = [sfan_v[pl.ds(jb + i * L, L)] for i in range(nsub)]
                perms = [sperm_v[pl.ds(jb + i * L, L)]
                         for i in range(nsub)]
                kmax = sgmax_v[pl.ds(gg + nsub - 1, L)][0]

                zero = jnp.zeros((L,), jnp.float32)
                init = zero if nsub == 1 else tuple(zero
                                                    for _ in range(nsub))

                @plsc.parallel_loop(0, kmax, unroll=unroll, carry=init)
                def max_body(kk, acc):
                    if nsub == 1:
                        return seg_block(kk, locs[0], fans[0], acc)
                    return tuple(
                        seg_block(kk, locs[i], fans[i], acc[i])
                        for i in range(nsub))
                accs = (max_body,) if nsub == 1 else max_body
                for i in range(nsub):
                    plsc.store_scatter(out_v, [perms[i]], accs[i])

            @plsc.parallel_loop(0, grp // 8, unroll=1)
            def seg_body(pg):
                macro_group(c * cseg + pg * (8 * L),
                            c * grp + pg * 8, 8, 1)

            macro_group(c * cseg + (grp - 1) * L,
                        c * grp + (grp - 1), 1, 4)

        start_dma(0, win0_v, sem0)

        def pair_body(c2, _):
            c0 = c2 * 2
            start_dma(c0 + 1, win1_v, sem1)
            wait_dma(win0_v, sem0)
            compute(c0, win0_v)
            start_dma(c0 + 2, win0_v, sem0)
            wait_dma(win1_v, sem1)
            compute(c0 + 1, win1_v)
            return 0
        lax.fori_loop(0, nchunk // 2, pair_body, 0)
        wait_dma(win0_v, sem0)
        compute(nchunk - 1, win0_v)
        pltpu.sync_copy(out_v, out_hbm.at[b])

    return k(ptr, p, sr_pad, sstart, sfan, sperm, sgmax, cbase)


def kernel(ptr, travvec, W, tensor_idx, relmask, groupbys, groupbymasks):
    B, E = ptr.shape
    T = tensor_idx.shape[0]
    assert B == NW

    p = _masked_softmax_tc(ptr, relmask, travvec, W)

    start = jnp.concatenate(
        [g[:, 0].astype(jnp.int32) for g in groupbys])
    fan = jnp.concatenate(
        [m.sum(axis=1).astype(jnp.int32) for m in groupbymasks])
    nseg = sum(g.shape[0] for g in groupbys)
    fmax = max(g.shape[1] for g in groupbys)

    cseg = 400
    assert nseg % cseg == 0 and cseg % L == 0
    wcap = _round_up(cseg * fmax + fmax + 8, L)

    assert E <= (1 << 14)
    sr = jnp.bitwise_or(
        jnp.left_shift(tensor_idx[:, 0].astype(jnp.int32), 14),
        tensor_idx[:, 1].astype(jnp.int32))
    sr_pad = jnp.pad(sr, (0, _round_up(T + wcap, L) - T))

    nchunk = nseg // cseg
    grp = cseg // L
    cbase = (start[::cseg] // 8 * 8).astype(jnp.int32)
    order = jnp.argsort(fan.reshape(nchunk, cseg), axis=1)
    sstart = jnp.take_along_axis(
        start.reshape(nchunk, cseg), order, axis=1).reshape(-1)
    sfan = jnp.take_along_axis(
        fan.reshape(nchunk, cseg), order, axis=1).reshape(-1)
    sperm = (order + jnp.arange(nchunk, dtype=jnp.int32)[:, None] * cseg
             ).astype(jnp.int32).reshape(-1)
    ngrp = nseg // L
    sgmax = jnp.max(sfan.reshape(ngrp, L), axis=1).astype(jnp.int32)
    sgmax = jnp.pad(sgmax, (0, _round_up(ngrp + L, L) - ngrp))
    cbase = jnp.pad(cbase, (0, _round_up(nchunk + L, L) - nchunk))

    return _segmax_sc(ptr, p, sr_pad, sstart, sfan, sperm, sgmax, cbase,
                      nseg=nseg, cseg=cseg, fmax=fmax, wcap=wcap)
